# baseline (speedup 1.0000x reference)
import jax
import jax.numpy as jnp
from jax.experimental import pallas as pl
from jax.experimental.pallas import tpu as pltpu

_ROWS = 16384
_COLS = 128
_CHUNK_ROWS = (1024, 2048, 6144, 4096, 2048, 1024)
_OFFSETS = tuple(sum(_CHUNK_ROWS[:i]) for i in range(len(_CHUNK_ROWS)))
_NCHUNKS = len(_CHUNK_ROWS)


def _copy_body(x_hbm, o_hbm, buf, in_sem, out_sem):
    def in_cp(i):
        off, n = _OFFSETS[i], _CHUNK_ROWS[i]
        return pltpu.make_async_copy(
            x_hbm.at[pl.ds(off, n), :], buf.at[pl.ds(off, n), :], in_sem.at[i]
        )

    def out_cp(i):
        off, n = _OFFSETS[i], _CHUNK_ROWS[i]
        return pltpu.make_async_copy(
            buf.at[pl.ds(off, n), :], o_hbm.at[pl.ds(off, n), :], out_sem.at[i]
        )

    for i in range(_NCHUNKS):
        in_cp(i).start()
    for i in range(_NCHUNKS):
        in_cp(i).wait()
        out_cp(i).start()
    for i in range(_NCHUNKS):
        out_cp(i).wait()


def kernel(x):
    return pl.pallas_call(
        _copy_body,
        in_specs=[pl.BlockSpec(memory_space=pl.ANY)],
        out_specs=pl.BlockSpec(memory_space=pl.ANY),
        scratch_shapes=[
            pltpu.VMEM((_ROWS, _COLS), jnp.float32),
            pltpu.SemaphoreType.DMA((_NCHUNKS,)),
            pltpu.SemaphoreType.DMA((_NCHUNKS,)),
        ],
        out_shape=jax.ShapeDtypeStruct(x.shape, x.dtype),
    )(x)


# speedup vs baseline: 1.0137x; 1.0137x over previous
import jax
import jax.numpy as jnp
from jax.experimental import pallas as pl
from jax.experimental.pallas import tpu as pltpu

_ROWS = 16384
_COLS = 128
_CHUNK_ROWS = (1024, 3072, 4096, 4096, 3072, 1024)
_OFFSETS = tuple(sum(_CHUNK_ROWS[:i]) for i in range(len(_CHUNK_ROWS)))
_NCHUNKS = len(_CHUNK_ROWS)


def _copy_body(x_hbm, o_hbm, buf, in_sem, out_sem):
    def in_cp(i):
        off, n = _OFFSETS[i], _CHUNK_ROWS[i]
        return pltpu.make_async_copy(
            x_hbm.at[pl.ds(off, n), :], buf.at[pl.ds(off, n), :], in_sem.at[i]
        )

    def out_cp(i):
        off, n = _OFFSETS[i], _CHUNK_ROWS[i]
        return pltpu.make_async_copy(
            buf.at[pl.ds(off, n), :], o_hbm.at[pl.ds(off, n), :], out_sem.at[i]
        )

    for i in range(_NCHUNKS):
        in_cp(i).start()
    for i in range(_NCHUNKS):
        in_cp(i).wait()
        out_cp(i).start()
    for i in range(_NCHUNKS):
        out_cp(i).wait()


def kernel(x):
    return pl.pallas_call(
        _copy_body,
        in_specs=[pl.BlockSpec(memory_space=pl.ANY)],
        out_specs=pl.BlockSpec(memory_space=pl.ANY),
        scratch_shapes=[
            pltpu.VMEM((_ROWS, _COLS), jnp.float32),
            pltpu.SemaphoreType.DMA((_NCHUNKS,)),
            pltpu.SemaphoreType.DMA((_NCHUNKS,)),
        ],
        out_shape=jax.ShapeDtypeStruct(x.shape, x.dtype),
    )(x)


# speedup vs baseline: 1.0185x; 1.0047x over previous
import jax
import jax.numpy as jnp
from jax.experimental import pallas as pl
from jax.experimental.pallas import tpu as pltpu

_ROWS = 16384
_COLS = 128
_CHUNK_ROWS = (1024, 2048, 5120, 5120, 2048, 1024)
_OFFSETS = tuple(sum(_CHUNK_ROWS[:i]) for i in range(len(_CHUNK_ROWS)))
_NCHUNKS = len(_CHUNK_ROWS)


def _copy_body(x_hbm, o_hbm, buf, in_sem, out_sem):
    def in_cp(i):
        off, n = _OFFSETS[i], _CHUNK_ROWS[i]
        return pltpu.make_async_copy(
            x_hbm.at[pl.ds(off, n), :], buf.at[pl.ds(off, n), :], in_sem.at[i]
        )

    def out_cp(i):
        off, n = _OFFSETS[i], _CHUNK_ROWS[i]
        return pltpu.make_async_copy(
            buf.at[pl.ds(off, n), :], o_hbm.at[pl.ds(off, n), :], out_sem.at[i]
        )

    for i in range(_NCHUNKS):
        in_cp(i).start()
    for i in range(_NCHUNKS):
        in_cp(i).wait()
        out_cp(i).start()
    for i in range(_NCHUNKS):
        out_cp(i).wait()


def kernel(x):
    return pl.pallas_call(
        _copy_body,
        in_specs=[pl.BlockSpec(memory_space=pl.ANY)],
        out_specs=pl.BlockSpec(memory_space=pl.ANY),
        scratch_shapes=[
            pltpu.VMEM((_ROWS, _COLS), jnp.float32),
            pltpu.SemaphoreType.DMA((_NCHUNKS,)),
            pltpu.SemaphoreType.DMA((_NCHUNKS,)),
        ],
        out_shape=jax.ShapeDtypeStruct(x.shape, x.dtype),
    )(x)
